# baseline (speedup 1.0000x reference)
import functools

import jax
import jax.numpy as jnp
from jax import lax
from jax.experimental import pallas as pl
from jax.experimental.pallas import tpu as pltpu
from jax.experimental.pallas import tpu_sc as plsc

B = 1024
D = 1024
E = 8
T = 12
TP = 16
OP = 128
BM = 128
NB = (2 * B + E * BM) // BM
P = NB * BM
NBPAD = 128
BME = 512

_NC, _NS, _L = 2, 16, 16
_NW = _NC * _NS
_TOKW = B // _NW


def _enc_body(x_ref, w1_ref, b1_ref, w2_ref, b2_ref, out_ref):
    x = x_ref[0]
    h = jnp.dot(x, w1_ref[0], preferred_element_type=jnp.float32) + b1_ref[0, 0]
    h = jnp.maximum(h, 0.0)
    o = jnp.dot(h, w2_ref[0], preferred_element_type=jnp.float32) + b2_ref[0, 0]
    n = jnp.sum(jnp.abs(o), axis=1, keepdims=True)
    out_ref[0] = o / jnp.maximum(n, 1e-12)


def _router_body(x_ref, wg_ref, d1_ref, d2_ref, g1_ref, g2_ref, be_ref, aux_ref):
    v = pl.program_id(0)
    x = x_ref[0]
    logits = jnp.dot(x, wg_ref[0], preferred_element_type=jnp.float32)
    m = jnp.max(logits, axis=-1, keepdims=True)
    ex = jnp.exp(logits - m)
    probs = ex / jnp.sum(ex, axis=-1, keepdims=True)

    triE = (lax.broadcasted_iota(jnp.int32, (E, E), 0)
            <= lax.broadcasted_iota(jnp.int32, (E, E), 1)).astype(jnp.float32)
    m1 = jnp.max(probs, axis=-1, keepdims=True)
    is1 = probs == m1
    cs1 = jnp.dot(is1.astype(jnp.float32), triE, preferred_element_type=jnp.float32)
    first1 = jnp.logical_and(is1, cs1 == 1.0).astype(jnp.float32)
    p2 = jnp.where(first1 > 0, -1.0, probs)
    m2 = jnp.max(p2, axis=-1, keepdims=True)
    is2 = p2 == m2
    cs2 = jnp.dot(is2.astype(jnp.float32), triE, preferred_element_type=jnp.float32)
    first2 = jnp.logical_and(is2, cs2 == 1.0).astype(jnp.float32)
    gsum = m1 + m2
    g1_ref[0] = jnp.broadcast_to(m1 / gsum, (B, TP))
    g2_ref[0] = jnp.broadcast_to(m2 / gsum, (B, TP))

    masks = jnp.concatenate([first1, first2], axis=-1)
    triB = (lax.broadcasted_iota(jnp.int32, (B, B), 0)
            > lax.broadcasted_iota(jnp.int32, (B, B), 1)).astype(jnp.float32)
    ranks = jnp.dot(triB, masks, preferred_element_type=jnp.float32)
    rank1 = ranks[:, :E]
    rank2 = ranks[:, E:]
    cnt1 = jnp.sum(first1, axis=0, keepdims=True)
    cnt2 = jnp.sum(first2, axis=0, keepdims=True)
    counts = cnt1 + cnt2
    pc = jnp.ceil(counts / BM) * BM
    striuE = (lax.broadcasted_iota(jnp.int32, (E, E), 0)
              < lax.broadcasted_iota(jnp.int32, (E, E), 1)).astype(jnp.float32)
    off = jnp.dot(pc, striuE, preferred_element_type=jnp.float32)

    vbase = v.astype(jnp.float32) * P
    dest1 = jnp.sum(first1 * (off + rank1), axis=-1) + vbase
    dest2 = jnp.sum(first2 * (off + cnt1 + rank2), axis=-1) + vbase
    d1_ref[0, 0] = dest1.astype(jnp.int32)
    d2_ref[0, 0] = dest2.astype(jnp.int32)

    p_iota = lax.broadcasted_iota(jnp.int32, (NBPAD, E), 0).astype(jnp.float32)
    ge = (p_iota >= (off / BM)).astype(jnp.float32)
    be_ref[0, 0] = (jnp.sum(ge, axis=-1) - 1.0).astype(jnp.int32)

    fe = jnp.mean(first1, axis=0)
    pe = jnp.mean(probs, axis=0)
    aux_ref[0, 0] = fe * pe


def _gemm_body(be_ref, x_ref, w1_ref, b1_ref, w2_ref, b2_ref, o_ref):
    del be_ref
    x = x_ref[0]
    h = jnp.dot(x, w1_ref[0, 0], preferred_element_type=jnp.float32) + b1_ref[0, 0, 0]
    h = jnp.maximum(h, 0.0)
    o_ref[0] = jnp.dot(h, w2_ref[0, 0], preferred_element_type=jnp.float32) + b2_ref[0, 0, 0]


def _sc_mesh():
    return plsc.VectorSubcoreMesh(core_axis_name="c", subcore_axis_name="s",
                                  num_cores=_NC, num_subcores=_NS)


def _sc_scatter(x_flat, d1f, d2f):
    def body(x_hbm, d1_hbm, d2_hbm, out_hbm, rows_v, i1_v, i2_v, sem):
        c = lax.axis_index("c")
        s = lax.axis_index("s")
        wid = s * _NC + c
        for v in range(4):
            tbase = v * B + wid * _TOKW
            pltpu.sync_copy(x_hbm.at[pl.ds(tbase, _TOKW)], rows_v)
            pltpu.sync_copy(d1_hbm.at[pl.ds(tbase, _TOKW)], i1_v)
            pltpu.sync_copy(d2_hbm.at[pl.ds(tbase, _TOKW)], i2_v)
            pltpu.async_copy(rows_v, out_hbm.at[i1_v], sem).wait()
            pltpu.async_copy(rows_v, out_hbm.at[i2_v], sem).wait()

    fn = pl.kernel(
        body,
        out_type=jax.ShapeDtypeStruct((4 * P, D), jnp.float32),
        mesh=_sc_mesh(),
        scratch_types=[
            pltpu.VMEM((_TOKW, D), jnp.float32),
            pltpu.VMEM((_TOKW,), jnp.int32),
            pltpu.VMEM((_TOKW,), jnp.int32),
            pltpu.SemaphoreType.DMA,
        ],
    )
    return fn(x_flat, d1f, d2f)


def _sc_combine(o_flat, d1f, d2f, g1f, g2f):
    def body(o_hbm, d1_hbm, d2_hbm, g1_hbm, g2_hbm, out_hbm,
             i1_v, i2_v, g1_v, g2_v, r1_v, r2_v, o_v, sem):
        c = lax.axis_index("c")
        s = lax.axis_index("s")
        wid = s * _NC + c
        for v in range(4):
            tbase = v * B + wid * _TOKW
            pltpu.sync_copy(d1_hbm.at[pl.ds(tbase, _TOKW)], i1_v)
            pltpu.sync_copy(d2_hbm.at[pl.ds(tbase, _TOKW)], i2_v)
            pltpu.sync_copy(g1_hbm.at[pl.ds(tbase, _TOKW)], g1_v)
            pltpu.sync_copy(g2_hbm.at[pl.ds(tbase, _TOKW)], g2_v)
            pltpu.async_copy(o_hbm.at[i1_v], r1_v, sem).wait()
            pltpu.async_copy(o_hbm.at[i2_v], r2_v, sem).wait()
            for i in range(_TOKW):
                o_v[i] = r1_v[i, pl.ds(0, TP)] * g1_v[i] + r2_v[i, pl.ds(0, TP)] * g2_v[i]
            pltpu.sync_copy(o_v, out_hbm.at[pl.ds(tbase, _TOKW)])

    fn = pl.kernel(
        body,
        out_type=jax.ShapeDtypeStruct((4 * B, TP), jnp.float32),
        mesh=_sc_mesh(),
        scratch_types=[
            pltpu.VMEM((_TOKW,), jnp.int32),
            pltpu.VMEM((_TOKW,), jnp.int32),
            pltpu.VMEM((_TOKW, TP), jnp.float32),
            pltpu.VMEM((_TOKW, TP), jnp.float32),
            pltpu.VMEM((_TOKW, OP), jnp.float32),
            pltpu.VMEM((_TOKW, OP), jnp.float32),
            pltpu.VMEM((_TOKW, TP), jnp.float32),
            pltpu.SemaphoreType.DMA,
        ],
    )
    return fn(o_flat, d1f, d2f, g1f, g2f)


def kernel(input_molecule, params):
    Xv = jnp.transpose(input_molecule, (1, 0, 2))[jnp.array([1, 2, 3, 0])]

    enc_order = (1, 2, 3, 0)
    w1e = jnp.stack([params['enc'][i]['w1'] for i in enc_order])
    b1e = jnp.stack([params['enc'][i]['b1'] for i in enc_order])[:, None, :]
    w2e = jnp.stack([params['enc'][i]['w2'] for i in enc_order])
    b2e = jnp.stack([params['enc'][i]['b2'] for i in enc_order])[:, None, :]

    wg = jnp.stack([p['wg'] for p in params['clf']])
    w1c = jnp.stack([p['w1'] for p in params['clf']])
    b1c = jnp.stack([p['b1'] for p in params['clf']])[:, :, None, :]
    w2c = jnp.stack([p['w2'] for p in params['clf']])
    w2c = jnp.pad(w2c, ((0, 0), (0, 0), (0, 0), (0, OP - T)))
    b2c = jnp.stack([p['b2'] for p in params['clf']])[:, :, None, :]
    b2c = jnp.pad(b2c, ((0, 0), (0, 0), (0, 0), (0, OP - T)))

    enc_out = pl.pallas_call(
        _enc_body,
        grid=(4, B // BME),
        in_specs=[
            pl.BlockSpec((1, BME, D), lambda v, m: (v, m, 0)),
            pl.BlockSpec((1, D, D), lambda v, m: (v, 0, 0)),
            pl.BlockSpec((1, 1, D), lambda v, m: (v, 0, 0)),
            pl.BlockSpec((1, D, D), lambda v, m: (v, 0, 0)),
            pl.BlockSpec((1, 1, D), lambda v, m: (v, 0, 0)),
        ],
        out_specs=pl.BlockSpec((1, BME, D), lambda v, m: (v, m, 0)),
        out_shape=jax.ShapeDtypeStruct((4, B, D), jnp.float32),
    )(Xv, w1e, b1e, w2e, b2e)

    d1, d2, g1, g2, be, aux = pl.pallas_call(
        _router_body,
        grid=(4,),
        in_specs=[
            pl.BlockSpec((1, B, D), lambda v: (v, 0, 0)),
            pl.BlockSpec((1, D, E), lambda v: (v, 0, 0)),
        ],
        out_specs=[
            pl.BlockSpec((1, 1, B), lambda v: (v, 0, 0)),
            pl.BlockSpec((1, 1, B), lambda v: (v, 0, 0)),
            pl.BlockSpec((1, B, TP), lambda v: (v, 0, 0)),
            pl.BlockSpec((1, B, TP), lambda v: (v, 0, 0)),
            pl.BlockSpec((1, 1, NBPAD), lambda v: (v, 0, 0)),
            pl.BlockSpec((1, 1, E), lambda v: (v, 0, 0)),
        ],
        out_shape=[
            jax.ShapeDtypeStruct((4, 1, B), jnp.int32),
            jax.ShapeDtypeStruct((4, 1, B), jnp.int32),
            jax.ShapeDtypeStruct((4, B, TP), jnp.float32),
            jax.ShapeDtypeStruct((4, B, TP), jnp.float32),
            jax.ShapeDtypeStruct((4, 1, NBPAD), jnp.int32),
            jax.ShapeDtypeStruct((4, 1, E), jnp.float32),
        ],
    )(Xv, wg)

    d1f = d1.reshape(4 * B)
    d2f = d2.reshape(4 * B)
    sorted_x = _sc_scatter(Xv.reshape(4 * B, D), d1f, d2f)

    o_sorted = pl.pallas_call(
        _gemm_body,
        grid_spec=pltpu.PrefetchScalarGridSpec(
            num_scalar_prefetch=1,
            grid=(4, NB),
            in_specs=[
                pl.BlockSpec((1, BM, D), lambda v, p, bt: (v, p, 0)),
                pl.BlockSpec((1, 1, D, D), lambda v, p, bt: (v, bt[v * NBPAD + p], 0, 0)),
                pl.BlockSpec((1, 1, 1, D), lambda v, p, bt: (v, bt[v * NBPAD + p], 0, 0)),
                pl.BlockSpec((1, 1, D, OP), lambda v, p, bt: (v, bt[v * NBPAD + p], 0, 0)),
                pl.BlockSpec((1, 1, 1, OP), lambda v, p, bt: (v, bt[v * NBPAD + p], 0, 0)),
            ],
            out_specs=pl.BlockSpec((1, BM, OP), lambda v, p, bt: (v, p, 0)),
        ),
        out_shape=jax.ShapeDtypeStruct((4, P, OP), jnp.float32),
    )(be.reshape(4 * NBPAD), sorted_x.reshape(4, P, D), w1c, b1c, w2c, b2c)

    moe_flat = _sc_combine(o_sorted.reshape(4 * P, OP), d1f, d2f,
                           g1.reshape(4 * B, TP), g2.reshape(4 * B, TP))
    moe_out = moe_flat.reshape(4, B, TP)[:, :, :T]

    loss_auc = E * jnp.sum(aux)
    return (moe_out[3], moe_out[0], moe_out[1], moe_out[2],
            enc_out[0], enc_out[1], enc_out[2], enc_out[3], loss_auc)


# speedup vs baseline: 1.4755x; 1.4755x over previous
import jax
import jax.numpy as jnp
from jax import lax
from jax.experimental import pallas as pl
from jax.experimental.pallas import tpu as pltpu
from jax.experimental.pallas import tpu_sc as plsc

B = 1024
D = 1024
E = 8
T = 12
TP = 16
OP = 128
BM = 128
NB = (2 * B + E * BM) // BM
P = NB * BM
NBPAD = 128
BME = 512

_NC, _NS, _L = 2, 16, 16
_NW = _NC * _NS
_TOKW = B // _NW

VIEW_ROWS = (1, 2, 3, 0)


def _enc_body(x_ref, w1_ref, b1_ref, w2_ref, b2_ref, out_ref):
    h = jnp.dot(x_ref[...], w1_ref[...], preferred_element_type=jnp.float32) + b1_ref[0]
    h = jnp.maximum(h, 0.0)
    o = jnp.dot(h, w2_ref[...], preferred_element_type=jnp.float32) + b2_ref[0]
    n = jnp.sum(jnp.abs(o), axis=1, keepdims=True)
    out_ref[...] = o / jnp.maximum(n, 1e-12)


def _router_body(x_ref, wg_ref, d1_ref, d2_ref, g1_ref, g2_ref, be_ref, aux_ref):
    x = x_ref[...]
    logits = jnp.dot(x, wg_ref[0], preferred_element_type=jnp.float32)
    m = jnp.max(logits, axis=-1, keepdims=True)
    ex = jnp.exp(logits - m)
    probs = ex / jnp.sum(ex, axis=-1, keepdims=True)

    triE = (lax.broadcasted_iota(jnp.int32, (E, E), 0)
            <= lax.broadcasted_iota(jnp.int32, (E, E), 1)).astype(jnp.float32)
    m1 = jnp.max(probs, axis=-1, keepdims=True)
    is1 = probs == m1
    cs1 = jnp.dot(is1.astype(jnp.float32), triE, preferred_element_type=jnp.float32)
    first1 = jnp.logical_and(is1, cs1 == 1.0).astype(jnp.float32)
    p2 = jnp.where(first1 > 0, -1.0, probs)
    m2 = jnp.max(p2, axis=-1, keepdims=True)
    is2 = p2 == m2
    cs2 = jnp.dot(is2.astype(jnp.float32), triE, preferred_element_type=jnp.float32)
    first2 = jnp.logical_and(is2, cs2 == 1.0).astype(jnp.float32)
    gsum = m1 + m2
    g1_ref[0] = jnp.broadcast_to(m1 / gsum, (B, TP))
    g2_ref[0] = jnp.broadcast_to(m2 / gsum, (B, TP))

    masks = jnp.concatenate([first1, first2], axis=-1)
    triB = (lax.broadcasted_iota(jnp.int32, (B, B), 0)
            > lax.broadcasted_iota(jnp.int32, (B, B), 1)).astype(jnp.float32)
    ranks = jnp.dot(triB, masks, preferred_element_type=jnp.float32)
    rank1 = ranks[:, :E]
    rank2 = ranks[:, E:]
    cnt1 = jnp.sum(first1, axis=0, keepdims=True)
    cnt2 = jnp.sum(first2, axis=0, keepdims=True)
    counts = cnt1 + cnt2
    pc = jnp.ceil(counts / BM) * BM
    striuE = (lax.broadcasted_iota(jnp.int32, (E, E), 0)
              < lax.broadcasted_iota(jnp.int32, (E, E), 1)).astype(jnp.float32)
    off = jnp.dot(pc, striuE, preferred_element_type=jnp.float32)

    dest1 = jnp.sum(first1 * (off + rank1), axis=-1)
    dest2 = jnp.sum(first2 * (off + cnt1 + rank2), axis=-1)
    d1_ref[0, 0] = dest1.astype(jnp.int32)
    d2_ref[0, 0] = dest2.astype(jnp.int32)

    p_iota = lax.broadcasted_iota(jnp.int32, (NBPAD, E), 0).astype(jnp.float32)
    ge = (p_iota >= (off / BM)).astype(jnp.float32)
    be_ref[0, 0] = (jnp.sum(ge, axis=-1) - 1.0).astype(jnp.int32)

    fe = jnp.mean(first1, axis=0)
    pe = jnp.mean(probs, axis=0)
    aux_ref[0, 0] = fe * pe


def _gemm_body(be_ref, x_ref, w1_ref, b1_ref, w2_ref, b2_ref, o_ref):
    del be_ref
    h = jnp.dot(x_ref[...], w1_ref[0], preferred_element_type=jnp.float32) + b1_ref[0, 0]
    h = jnp.maximum(h, 0.0)
    o = jnp.dot(h, w2_ref[0], preferred_element_type=jnp.float32) + b2_ref[0, 0]
    o_ref[...] = jnp.concatenate([o, jnp.zeros((BM, OP - T), jnp.float32)], axis=1)


def _sc_mesh():
    return plsc.VectorSubcoreMesh(core_axis_name="c", subcore_axis_name="s",
                                  num_cores=_NC, num_subcores=_NS)


def _sc_dispatch(x_flat, d1f, d2f):
    def body(x_hbm, d1_hbm, d2_hbm, o0, o1, o2, o3,
             idx_v, rows_v, i1_v, i2_v, sem):
        c = lax.axis_index("c")
        s = lax.axis_index("s")
        wid = s * _NC + c
        outs = (o0, o1, o2, o3)
        lane = lax.broadcasted_iota(jnp.int32, (_L,), 0)
        for v in range(4):
            vrow = VIEW_ROWS[v]
            for j in range(_TOKW // _L):
                base = 4 * (wid * _TOKW + j * _L) + vrow
                idx_v[pl.ds(j * _L, _L)] = base + 4 * lane
            tbase = v * B + wid * _TOKW
            pltpu.sync_copy(d1_hbm.at[pl.ds(tbase, _TOKW)], i1_v)
            pltpu.sync_copy(d2_hbm.at[pl.ds(tbase, _TOKW)], i2_v)
            pltpu.async_copy(x_hbm.at[idx_v], rows_v, sem).wait()
            pltpu.async_copy(rows_v, outs[v].at[i1_v], sem).wait()
            pltpu.async_copy(rows_v, outs[v].at[i2_v], sem).wait()

    fn = pl.kernel(
        body,
        out_type=[jax.ShapeDtypeStruct((P, D), jnp.float32) for _ in range(4)],
        mesh=_sc_mesh(),
        scratch_types=[
            pltpu.VMEM((_TOKW,), jnp.int32),
            pltpu.VMEM((_TOKW, D), jnp.float32),
            pltpu.VMEM((_TOKW,), jnp.int32),
            pltpu.VMEM((_TOKW,), jnp.int32),
            pltpu.SemaphoreType.DMA,
        ],
    )
    return fn(x_flat, d1f, d2f)


def _sc_combine(o_list, d1f, d2f, g1f, g2f):
    def body(o0, o1, o2, o3, d1_hbm, d2_hbm, g1_hbm, g2_hbm,
             u0, u1, u2, u3, i1_v, i2_v, g1_v, g2_v, r1_v, r2_v, o_v, sem):
        c = lax.axis_index("c")
        s = lax.axis_index("s")
        wid = s * _NC + c
        tabs = (o0, o1, o2, o3)
        outs = (u0, u1, u2, u3)
        for v in range(4):
            tbase = v * B + wid * _TOKW
            pltpu.sync_copy(d1_hbm.at[pl.ds(tbase, _TOKW)], i1_v)
            pltpu.sync_copy(d2_hbm.at[pl.ds(tbase, _TOKW)], i2_v)
            pltpu.sync_copy(g1_hbm.at[pl.ds(tbase, _TOKW)], g1_v)
            pltpu.sync_copy(g2_hbm.at[pl.ds(tbase, _TOKW)], g2_v)
            pltpu.async_copy(tabs[v].at[i1_v], r1_v, sem).wait()
            pltpu.async_copy(tabs[v].at[i2_v], r2_v, sem).wait()
            for i in range(_TOKW):
                o_v[i] = r1_v[i, pl.ds(0, TP)] * g1_v[i] + r2_v[i, pl.ds(0, TP)] * g2_v[i]
            pltpu.sync_copy(o_v, outs[v].at[pl.ds(wid * _TOKW, _TOKW)])

    fn = pl.kernel(
        body,
        out_type=[jax.ShapeDtypeStruct((B, TP), jnp.float32) for _ in range(4)],
        mesh=_sc_mesh(),
        scratch_types=[
            pltpu.VMEM((_TOKW,), jnp.int32),
            pltpu.VMEM((_TOKW,), jnp.int32),
            pltpu.VMEM((_TOKW, TP), jnp.float32),
            pltpu.VMEM((_TOKW, TP), jnp.float32),
            pltpu.VMEM((_TOKW, OP), jnp.float32),
            pltpu.VMEM((_TOKW, OP), jnp.float32),
            pltpu.VMEM((_TOKW, TP), jnp.float32),
            pltpu.SemaphoreType.DMA,
        ],
    )
    return fn(o_list[0], o_list[1], o_list[2], o_list[3], d1f, d2f, g1f, g2f)


def kernel(input_molecule, params):
    XR = input_molecule.reshape(B, 4 * D)
    x_flat = input_molecule.reshape(4 * B, D)

    wg = jnp.stack([p['wg'] for p in params['clf']])

    enc_out = []
    for v in range(4):
        ep = params['enc'][VIEW_ROWS[v]]
        enc_out.append(pl.pallas_call(
            _enc_body,
            grid=(B // BME,),
            in_specs=[
                pl.BlockSpec((BME, D), lambda m, r=VIEW_ROWS[v]: (m, r)),
                pl.BlockSpec((D, D), lambda m: (0, 0)),
                pl.BlockSpec((1, D), lambda m: (0, 0)),
                pl.BlockSpec((D, D), lambda m: (0, 0)),
                pl.BlockSpec((1, D), lambda m: (0, 0)),
            ],
            out_specs=pl.BlockSpec((BME, D), lambda m: (m, 0)),
            out_shape=jax.ShapeDtypeStruct((B, D), jnp.float32),
        )(XR, ep['w1'], ep['b1'].reshape(1, D), ep['w2'], ep['b2'].reshape(1, D)))

    d1, d2, g1, g2, be, aux = pl.pallas_call(
        _router_body,
        grid=(4,),
        in_specs=[
            pl.BlockSpec((B, D), lambda v: (0, lax.rem(v + 1, 4))),
            pl.BlockSpec((1, D, E), lambda v: (v, 0, 0)),
        ],
        out_specs=[
            pl.BlockSpec((1, 1, B), lambda v: (v, 0, 0)),
            pl.BlockSpec((1, 1, B), lambda v: (v, 0, 0)),
            pl.BlockSpec((1, B, TP), lambda v: (v, 0, 0)),
            pl.BlockSpec((1, B, TP), lambda v: (v, 0, 0)),
            pl.BlockSpec((1, 1, NBPAD), lambda v: (v, 0, 0)),
            pl.BlockSpec((1, 1, E), lambda v: (v, 0, 0)),
        ],
        out_shape=[
            jax.ShapeDtypeStruct((4, 1, B), jnp.int32),
            jax.ShapeDtypeStruct((4, 1, B), jnp.int32),
            jax.ShapeDtypeStruct((4, B, TP), jnp.float32),
            jax.ShapeDtypeStruct((4, B, TP), jnp.float32),
            jax.ShapeDtypeStruct((4, 1, NBPAD), jnp.int32),
            jax.ShapeDtypeStruct((4, 1, E), jnp.float32),
        ],
    )(XR, wg)

    d1f = d1.reshape(4 * B)
    d2f = d2.reshape(4 * B)
    sorted_xs = _sc_dispatch(x_flat, d1f, d2f)

    be_flat = be.reshape(4 * NBPAD)
    o_list = []
    for v in range(4):
        cp = params['clf'][v]
        o_list.append(pl.pallas_call(
            _gemm_body,
            grid_spec=pltpu.PrefetchScalarGridSpec(
                num_scalar_prefetch=1,
                grid=(NB,),
                in_specs=[
                    pl.BlockSpec((BM, D), lambda p, bt: (p, 0)),
                    pl.BlockSpec((1, D, D), lambda p, bt, v0=v: (bt[v0 * NBPAD + p], 0, 0)),
                    pl.BlockSpec((1, 1, D), lambda p, bt, v0=v: (bt[v0 * NBPAD + p], 0, 0)),
                    pl.BlockSpec((1, D, T), lambda p, bt, v0=v: (bt[v0 * NBPAD + p], 0, 0)),
                    pl.BlockSpec((1, 1, T), lambda p, bt, v0=v: (bt[v0 * NBPAD + p], 0, 0)),
                ],
                out_specs=pl.BlockSpec((BM, OP), lambda p, bt: (p, 0)),
            ),
            out_shape=jax.ShapeDtypeStruct((P, OP), jnp.float32),
        )(be_flat, sorted_xs[v], cp['w1'], cp['b1'].reshape(E, 1, D),
          cp['w2'], cp['b2'].reshape(E, 1, T)))

    moe = _sc_combine(o_list, d1f, d2f, g1.reshape(4 * B, TP), g2.reshape(4 * B, TP))
    moe = [m[:, :T] for m in moe]

    loss_auc = E * jnp.sum(aux)
    return (moe[3], moe[0], moe[1], moe[2],
            enc_out[0], enc_out[1], enc_out[2], enc_out[3], loss_auc)
